# baseline (speedup 1.0000x reference)
import functools

import jax
import jax.numpy as jnp
from jax.experimental import pallas as pl
from jax.experimental.pallas import tpu as pltpu


def _half_body(we_ref, x_ref, wg_ref, lt_ref, w4_ref, *, kwe, kpe):
    k = pl.program_id(1)

    @pl.when(k == 0)
    def _init():
        w4_ref[0:2] = jnp.transpose(wg_ref[...])
        w4_ref[2:4] = jnp.zeros_like(w4_ref[2:4])

    @pl.when(k < kwe)
    def _accum():
        e = k // kpe
        w4_ref[pl.ds(2 + e, 1)] += jnp.sum(we_ref[0], axis=0, keepdims=True)

    @pl.when(k >= kwe)
    def _tokens():
        lt_ref[0] = jax.lax.dot_general(
            x_ref[...], w4_ref[...],
            dimension_numbers=(((1,), (1,)), ((), ())),
            preferred_element_type=jnp.float32)


def _route_body(ltp_ref, be_ref, sl_ref, out_ref, cnt_ref, *, cap, t):
    c = pl.program_id(0)

    @pl.when(c == 0)
    def _init():
        cnt_ref[0] = 0.0

    lt = ltp_ref[0] + ltp_ref[1]
    l0, l1, t0, t1 = lt[:, 0:1], lt[:, 1:2], lt[:, 2:3], lt[:, 3:4]
    e1 = l1 > l0
    m = e1.astype(jnp.float32)

    excl = jnp.dot(sl_ref[...], m, preferred_element_type=jnp.float32)
    pos1 = excl + cnt_ref[0]
    slin = (c * t + jax.lax.broadcasted_iota(jnp.int32, (t, 1), 0)
            ).astype(jnp.float32)
    pos0 = slin - pos1
    pos = jnp.where(e1, pos1, pos0)
    within = (pos < cap).astype(jnp.float32)

    gate = jax.nn.sigmoid(jnp.abs(l1 - l0))
    bs0 = jnp.sum(be_ref[0:1, :])
    bs1 = jnp.sum(be_ref[1:2, :])
    tsel = jnp.where(e1, t1 + bs1, t0 + bs0)
    out_ref[...] = gate * within * tsel
    cnt_ref[0] += jnp.sum(m)


def _lsm_body(v_ref, out_ref):
    v = v_ref[...]
    mx = jnp.max(v, axis=1, keepdims=True)
    lse = jnp.log(jnp.sum(jnp.exp(v - mx), axis=1, keepdims=True)) + mx
    out_ref[...] = v - lse


def kernel(input, wg, We, be):
    B, SEQ, D = input.shape
    E = wg.shape[1]
    S = B * SEQ
    cap = -(-S // E)
    H = 2
    D2 = D // H

    x = input.reshape(S, D)

    F = 512
    KPE = D // F
    KWE = E * KPE
    T = 512
    C = S // T

    lt_parts = pl.pallas_call(
        functools.partial(_half_body, kwe=KWE, kpe=KPE),
        grid=(H, KWE + C),
        in_specs=[
            pl.BlockSpec((1, F, D2),
                         lambda h, k: (jnp.minimum(k, KWE - 1) // KPE,
                                       jnp.minimum(k, KWE - 1) % KPE, h)),
            pl.BlockSpec((T, D2), lambda h, k: (jnp.maximum(k - KWE, 0), h)),
            pl.BlockSpec((D2, E), lambda h, k: (h, 0)),
        ],
        out_specs=pl.BlockSpec(
            (1, T, 4), lambda h, k: (h, jnp.maximum(k - KWE, 0), 0)),
        out_shape=jax.ShapeDtypeStruct((H, S, 4), jnp.float32),
        scratch_shapes=[pltpu.VMEM((4, D2), jnp.float32)],
        compiler_params=pltpu.CompilerParams(
            dimension_semantics=("parallel", "arbitrary")),
    )(We, x, wg)

    ii = jax.lax.broadcasted_iota(jnp.int32, (T, T), 0)
    jj = jax.lax.broadcasted_iota(jnp.int32, (T, T), 1)
    sl = (jj < ii).astype(jnp.float32)

    val = pl.pallas_call(
        functools.partial(_route_body, cap=float(cap), t=T),
        grid=(C,),
        in_specs=[
            pl.BlockSpec((H, T, 4), lambda c: (0, c, 0)),
            pl.BlockSpec((E, D), lambda c: (0, 0)),
            pl.BlockSpec((T, T), lambda c: (0, 0)),
        ],
        out_specs=pl.BlockSpec((T, 1), lambda c: (c, 0)),
        out_shape=jax.ShapeDtypeStruct((S, 1), jnp.float32),
        scratch_shapes=[pltpu.SMEM((1,), jnp.float32)],
    )(lt_parts, be, sl)

    v = val.reshape(B, SEQ)

    out = pl.pallas_call(
        _lsm_body,
        in_specs=[pl.BlockSpec((B, SEQ), lambda: (0, 0))],
        out_specs=pl.BlockSpec((B, SEQ), lambda: (0, 0)),
        out_shape=jax.ShapeDtypeStruct((B, SEQ), jnp.float32),
    )(v)
    return out


# speedup vs baseline: 1.2338x; 1.2338x over previous
import functools

import jax
import jax.numpy as jnp
from jax.experimental import pallas as pl
from jax.experimental.pallas import tpu as pltpu


def _wsum_body(we_ref, out_ref, *, kpe2):
    j = pl.program_id(1)

    @pl.when(j % kpe2 == 0)
    def _init():
        out_ref[...] = jnp.zeros_like(out_ref)

    out_ref[0, 0] += jnp.sum(we_ref[0], axis=0, keepdims=True)


def _dots_body(x_ref, w4_ref, lt_ref):
    lt_ref[...] = jax.lax.dot_general(
        x_ref[...], w4_ref[...],
        dimension_numbers=(((1,), (1,)), ((), ())),
        preferred_element_type=jnp.float32)


def _route_body(lt_ref, be_ref, sl_ref, out_ref, cnt_ref, *, cap, t):
    c = pl.program_id(0)

    @pl.when(c == 0)
    def _init():
        cnt_ref[0] = 0.0

    lt = lt_ref[...]
    l0, l1, t0, t1 = lt[:, 0:1], lt[:, 1:2], lt[:, 2:3], lt[:, 3:4]
    e1 = l1 > l0
    m = e1.astype(jnp.float32)

    excl = jnp.dot(sl_ref[...], m, preferred_element_type=jnp.float32)
    pos1 = excl + cnt_ref[0]
    slin = (c * t + jax.lax.broadcasted_iota(jnp.int32, (t, 1), 0)
            ).astype(jnp.float32)
    pos0 = slin - pos1
    pos = jnp.where(e1, pos1, pos0)
    within = (pos < cap).astype(jnp.float32)

    gate = jax.nn.sigmoid(jnp.abs(l1 - l0))
    bs0 = jnp.sum(be_ref[0:1, :])
    bs1 = jnp.sum(be_ref[1:2, :])
    tsel = jnp.where(e1, t1 + bs1, t0 + bs0)
    out_ref[...] = gate * within * tsel
    cnt_ref[0] += jnp.sum(m)


def _lsm_body(v_ref, out_ref):
    v = v_ref[...]
    mx = jnp.max(v, axis=1, keepdims=True)
    lse = jnp.log(jnp.sum(jnp.exp(v - mx), axis=1, keepdims=True)) + mx
    out_ref[...] = v - lse


def kernel(input, wg, We, be):
    B, SEQ, D = input.shape
    E = wg.shape[1]
    S = B * SEQ
    cap = -(-S // E)
    H = 2

    x = input.reshape(S, D)

    F = 512
    KPE = D // F
    KPE2 = KPE // H
    T = 512
    C = S // T

    wsum_p = pl.pallas_call(
        functools.partial(_wsum_body, kpe2=KPE2),
        grid=(H, E * KPE2),
        in_specs=[
            pl.BlockSpec((1, F, D),
                         lambda h, j: (j // KPE2, h * KPE2 + j % KPE2, 0)),
        ],
        out_specs=pl.BlockSpec((1, 1, 1, D), lambda h, j: (h, j // KPE2, 0, 0)),
        out_shape=jax.ShapeDtypeStruct((H, E, 1, D), jnp.float32),
        compiler_params=pltpu.CompilerParams(
            dimension_semantics=("parallel", "arbitrary")),
    )(We)

    wsum = wsum_p.sum(axis=0).reshape(E, D)
    w4r = jnp.concatenate([wg.T, wsum], axis=0)

    lt = pl.pallas_call(
        _dots_body,
        grid=(H, C // H),
        in_specs=[
            pl.BlockSpec((T, D), lambda h, c: (h * (C // H) + c, 0)),
            pl.BlockSpec((4, D), lambda h, c: (0, 0)),
        ],
        out_specs=pl.BlockSpec((T, 4), lambda h, c: (h * (C // H) + c, 0)),
        out_shape=jax.ShapeDtypeStruct((S, 4), jnp.float32),
        compiler_params=pltpu.CompilerParams(
            dimension_semantics=("parallel", "arbitrary")),
    )(x, w4r)

    ii = jax.lax.broadcasted_iota(jnp.int32, (T, T), 0)
    jj = jax.lax.broadcasted_iota(jnp.int32, (T, T), 1)
    sl = (jj < ii).astype(jnp.float32)

    val = pl.pallas_call(
        functools.partial(_route_body, cap=float(cap), t=T),
        grid=(C,),
        in_specs=[
            pl.BlockSpec((T, 4), lambda c: (c, 0)),
            pl.BlockSpec((E, D), lambda c: (0, 0)),
            pl.BlockSpec((T, T), lambda c: (0, 0)),
        ],
        out_specs=pl.BlockSpec((T, 1), lambda c: (c, 0)),
        out_shape=jax.ShapeDtypeStruct((S, 1), jnp.float32),
        scratch_shapes=[pltpu.SMEM((1,), jnp.float32)],
    )(lt, be, sl)

    v = val.reshape(B, SEQ)

    out = pl.pallas_call(
        _lsm_body,
        in_specs=[pl.BlockSpec((B, SEQ), lambda: (0, 0))],
        out_specs=pl.BlockSpec((B, SEQ), lambda: (0, 0)),
        out_shape=jax.ShapeDtypeStruct((B, SEQ), jnp.float32),
    )(v)
    return out


# speedup vs baseline: 1.3772x; 1.1162x over previous
import functools

import jax
import jax.numpy as jnp
from jax.experimental import pallas as pl
from jax.experimental.pallas import tpu as pltpu


def _fused_body(we_ref, x_ref, wg_ref, be_ref, sl_ref, out_ref,
                w4_ref, cnt_ref, *, cap, t, kwe, kpe):
    k = pl.program_id(0)

    @pl.when(k == 0)
    def _init():
        w4_ref[0:2] = jnp.transpose(wg_ref[...])
        w4_ref[2:4] = jnp.zeros_like(w4_ref[2:4])
        cnt_ref[0] = 0.0

    @pl.when(k < kwe)
    def _accum():
        e = k // kpe
        part = jnp.sum(we_ref[0], axis=0, keepdims=True)
        w4_ref[pl.ds(2 + e, 1)] += part

    @pl.when(k >= kwe)
    def _tokens():
        c = k - kwe
        lt = jax.lax.dot_general(
            x_ref[...], w4_ref[...],
            dimension_numbers=(((1,), (1,)), ((), ())),
            preferred_element_type=jnp.float32)
        l0, l1, t0, t1 = lt[:, 0:1], lt[:, 1:2], lt[:, 2:3], lt[:, 3:4]
        e1 = l1 > l0
        m = e1.astype(jnp.float32)

        excl = jnp.dot(sl_ref[...], m, preferred_element_type=jnp.float32)
        pos1 = excl + cnt_ref[0]
        slin = (c * t + jax.lax.broadcasted_iota(jnp.int32, (t, 1), 0)
                ).astype(jnp.float32)
        pos0 = slin - pos1
        pos = jnp.where(e1, pos1, pos0)
        within = (pos < cap).astype(jnp.float32)

        gate = jax.nn.sigmoid(jnp.abs(l1 - l0))
        bs0 = jnp.sum(be_ref[0:1, :])
        bs1 = jnp.sum(be_ref[1:2, :])
        tsel = jnp.where(e1, t1 + bs1, t0 + bs0)
        out_ref[...] = gate * within * tsel
        cnt_ref[0] += jnp.sum(m)


def _lsm_body(v_ref, out_ref):
    v = v_ref[...]
    mx = jnp.max(v, axis=1, keepdims=True)
    lse = jnp.log(jnp.sum(jnp.exp(v - mx), axis=1, keepdims=True)) + mx
    out_ref[...] = v - lse


def kernel(input, wg, We, be):
    B, SEQ, D = input.shape
    E = wg.shape[1]
    S = B * SEQ
    cap = -(-S // E)

    x = input.reshape(S, D)

    F = 1024
    KPE = D // F
    KWE = E * KPE
    T = 1024
    C = S // T

    ii = jax.lax.broadcasted_iota(jnp.int32, (T, T), 0)
    jj = jax.lax.broadcasted_iota(jnp.int32, (T, T), 1)
    sl = (jj < ii).astype(jnp.float32)

    val = pl.pallas_call(
        functools.partial(_fused_body, cap=float(cap), t=T, kwe=KWE, kpe=KPE),
        grid=(KWE + C,),
        in_specs=[
            pl.BlockSpec((1, F, D),
                         lambda k: (jnp.minimum(k, KWE - 1) // KPE,
                                    jnp.minimum(k, KWE - 1) % KPE, 0)),
            pl.BlockSpec((T, D), lambda k: (jnp.maximum(k - KWE, 0), 0)),
            pl.BlockSpec((D, E), lambda k: (0, 0)),
            pl.BlockSpec((E, D), lambda k: (0, 0)),
            pl.BlockSpec((T, T), lambda k: (0, 0)),
        ],
        out_specs=pl.BlockSpec((T, 1), lambda k: (jnp.maximum(k - KWE, 0), 0)),
        out_shape=jax.ShapeDtypeStruct((S, 1), jnp.float32),
        scratch_shapes=[
            pltpu.VMEM((4, D), jnp.float32),
            pltpu.SMEM((1,), jnp.float32),
        ],
    )(We, x, wg, be, sl)

    v = val.reshape(B, SEQ)

    out = pl.pallas_call(
        _lsm_body,
        in_specs=[pl.BlockSpec((B, SEQ), lambda: (0, 0))],
        out_specs=pl.BlockSpec((B, SEQ), lambda: (0, 0)),
        out_shape=jax.ShapeDtypeStruct((B, SEQ), jnp.float32),
    )(v)
    return out


# speedup vs baseline: 1.5352x; 1.1147x over previous
import functools

import jax
import jax.numpy as jnp
from jax.experimental import pallas as pl
from jax.experimental.pallas import tpu as pltpu


def _fused_body(we_ref, x_ref, wg_ref, be_ref, su_ref, out_ref,
                w4_ref, cnt_ref, bs_ref, *, cap, t, kwe, kpe):
    k = pl.program_id(0)

    @pl.when(k == 0)
    def _init():
        w4_ref[0:2] = jnp.transpose(wg_ref[...])
        w4_ref[2:4] = jnp.zeros_like(w4_ref[2:4])
        cnt_ref[0] = 0.0
        bs_ref[0] = jnp.sum(be_ref[0:1, :])
        bs_ref[1] = jnp.sum(be_ref[1:2, :])

    @pl.when(k < kwe)
    def _accum():
        e = k // kpe
        part = jnp.sum(we_ref[0], axis=0, keepdims=True)
        w4_ref[pl.ds(2 + e, 1)] += part

    @pl.when(k >= kwe)
    def _tokens():
        c = k - kwe
        lt = jax.lax.dot_general(
            w4_ref[...], x_ref[...],
            dimension_numbers=(((1,), (1,)), ((), ())),
            preferred_element_type=jnp.float32)
        l0, l1, t0, t1 = lt[0:1], lt[1:2], lt[2:3], lt[3:4]
        e1 = l1 > l0
        m = e1.astype(jnp.float32)

        excl = jnp.dot(m, su_ref[...], preferred_element_type=jnp.float32)
        pos1 = excl + cnt_ref[0]
        slin = (c * t + jax.lax.broadcasted_iota(jnp.int32, (1, t), 1)
                ).astype(jnp.float32)
        pos0 = slin - pos1
        pos = jnp.where(e1, pos1, pos0)
        within = (pos < cap).astype(jnp.float32)

        gate = jax.nn.sigmoid(jnp.abs(l1 - l0))
        tsel = jnp.where(e1, t1 + bs_ref[1], t0 + bs_ref[0])
        out_ref[...] = gate * within * tsel
        cnt_ref[0] += jnp.sum(m)


def _lsm_body(v_ref, out_ref):
    v = v_ref[...]
    mx = jnp.max(v, axis=1, keepdims=True)
    lse = jnp.log(jnp.sum(jnp.exp(v - mx), axis=1, keepdims=True)) + mx
    out_ref[...] = v - lse


def kernel(input, wg, We, be):
    B, SEQ, D = input.shape
    E = wg.shape[1]
    S = B * SEQ
    cap = -(-S // E)

    x = input.reshape(S, D)

    F = 512
    KPE = D // F
    KWE = E * KPE
    T = 512
    C = S // T

    ii = jax.lax.broadcasted_iota(jnp.int32, (T, T), 0)
    jj = jax.lax.broadcasted_iota(jnp.int32, (T, T), 1)
    su = (ii < jj).astype(jnp.float32)

    val = pl.pallas_call(
        functools.partial(_fused_body, cap=float(cap), t=T, kwe=KWE, kpe=KPE),
        grid=(KWE + C,),
        in_specs=[
            pl.BlockSpec((1, F, D),
                         lambda k: (jnp.minimum(k, KWE - 1) // KPE,
                                    jnp.minimum(k, KWE - 1) % KPE, 0)),
            pl.BlockSpec((T, D), lambda k: (jnp.maximum(k - KWE, 0), 0)),
            pl.BlockSpec((D, E), lambda k: (0, 0)),
            pl.BlockSpec((E, D), lambda k: (0, 0)),
            pl.BlockSpec((T, T), lambda k: (0, 0)),
        ],
        out_specs=pl.BlockSpec((1, T), lambda k: (0, jnp.maximum(k - KWE, 0))),
        out_shape=jax.ShapeDtypeStruct((1, S), jnp.float32),
        scratch_shapes=[
            pltpu.VMEM((4, D), jnp.float32),
            pltpu.SMEM((1,), jnp.float32),
            pltpu.SMEM((2,), jnp.float32),
        ],
    )(We, x, wg, be, su)

    v = val.reshape(B, SEQ)

    out = pl.pallas_call(
        _lsm_body,
        in_specs=[pl.BlockSpec((B, SEQ), lambda: (0, 0))],
        out_specs=pl.BlockSpec((B, SEQ), lambda: (0, 0)),
        out_shape=jax.ShapeDtypeStruct((B, SEQ), jnp.float32),
    )(v)
    return out
